# baseline (speedup 1.0000x reference)
import functools

import jax
import jax.numpy as jnp
from jax import lax
from jax.experimental import pallas as pl
from jax.experimental.pallas import tpu as pltpu
from jax.experimental.pallas import tpu_sc as plsc

_N = 10000
_NP = 10240
_E = 320000
_D = 128
_NC = 2
_NS = 16
_NW = _NC * _NS
_EPW = _E // _NW
_EPT = _E // _NS
_K = 80
_NB = _EPW // _K
_NBT = _EPT // _K
_RPT = _NP // _NS
_DH = _D // 2

_mesh = plsc.VectorSubcoreMesh(core_axis_name="c", subcore_axis_name="s")
_sc_params = pltpu.CompilerParams(use_tc_tiling_on_sc=False)


@functools.partial(
    pl.kernel,
    out_type=jax.ShapeDtypeStruct((6, _NP, _D), jnp.float32),
    mesh=_mesh,
    scratch_types=[
        pltpu.VMEM_SHARED((_NP, 16), jnp.float32),
        pltpu.VMEM_SHARED((_NP, 16), jnp.float32),
        pltpu.VMEM((_EPW,), jnp.int32),
        pltpu.VMEM((_K, 16), jnp.float32),
        pltpu.SemaphoreType.DMA,
    ],
    compiler_params=_sc_params,
)
def _deg_kernel(idx_all, zeros_hbm, ones_hbm, out_hbm,
                sp0, sp1, iv, ones_v, sem):
    cid = lax.axis_index("c")
    sid = lax.axis_index("s")
    ebase = (cid * _NS + sid) * _EPW
    rbase = sid * _RPT
    sps = [sp0, sp1]
    pltpu.sync_copy(ones_hbm, ones_v)
    for g in range(3):
        for j in range(2):
            pltpu.sync_copy(zeros_hbm, sps[j].at[pl.ds(rbase, _RPT)])
        plsc.subcore_barrier()
        for j in range(2):
            h = g * 2 + j
            pltpu.sync_copy(idx_all.at[pl.ds(h * _E + ebase, _EPW)], iv)

            def fire(b, carry, j=j):
                pltpu.async_copy(ones_v, sps[j].at[iv.at[pl.ds(b * _K, _K)]],
                                 sem, add=True)
                return carry
            lax.fori_loop(0, _NB, fire, 0)

            def drain(b, carry):
                pltpu.make_async_copy(ones_hbm, ones_v, sem).wait()
                return carry
            lax.fori_loop(0, _NB, drain, 0)
        plsc.subcore_barrier()
        for j in range(2):
            pltpu.sync_copy(
                sps[j].at[pl.ds(rbase, _RPT)],
                out_hbm.at[g * 2 + j, pl.ds(rbase, _RPT),
                           pl.ds(cid * 16, 16)])
        plsc.subcore_barrier()


def _matmul3(X, degs, Ws2):
    blk = 1000

    def body(x_ref, d_ref, w_ref, o_ref):
        deg = d_ref[0, :, 0] + d_ref[0, :, 16]
        s = lax.rsqrt(jnp.maximum(deg, 1.0))
        x = x_ref[...] * s[:, None]
        o_ref[...] = jnp.dot(x, w_ref[0, 0],
                             preferred_element_type=jnp.float32)

    return pl.pallas_call(
        body,
        grid=(2, 3, _N // blk),
        in_specs=[
            pl.BlockSpec((blk, _D), lambda c, v, i: (i, 0)),
            pl.BlockSpec((1, blk, _D), lambda c, v, i: (v, i, 0)),
            pl.BlockSpec((1, 1, _D, _DH), lambda c, v, i: (c, v, 0, 0)),
        ],
        out_specs=pl.BlockSpec(
            (blk, _DH), lambda c, v, i: (c * 3 * (_N // 1000) +
                                         v * (_N // 1000) + i, 0)),
        out_shape=jax.ShapeDtypeStruct((2 * 3 * _N, _DH), jnp.float32),
    )(X, degs, Ws2)


_NBUF = 5
_NBH = _NBT // 2
_NG = _NBH // _NBUF
_NPAIR = (_NG - 1) // 2


@functools.partial(
    pl.kernel,
    out_type=jax.ShapeDtypeStruct((3, _NP, _D), jnp.float32),
    mesh=_mesh,
    scratch_types=[
        pltpu.VMEM_SHARED((_NP, _DH), jnp.float32),
        pltpu.VMEM((_NBH * _K,), jnp.int32),
        pltpu.VMEM((_NBH * _K,), jnp.int32),
        [pltpu.VMEM((_K, _DH), jnp.float32) for _ in range(2 * _NBUF)],
        [pltpu.SemaphoreType.DMA for _ in range(4)],
    ],
    compiler_params=_sc_params,
)
def _scatter_kernel(h2_hbm, idx_all, zeros_hbm, out_hbm,
                    agg, sv, dv, rows, sems):
    cid = lax.axis_index("c")
    sid = lax.axis_index("s")
    rbase = sid * _RPT
    sem_g = [sems[0], sems[1]]
    sem_s = [sems[2], sems[3]]

    def fire_g(tbl, g0, bank):
        for j in range(_NBUF):
            pltpu.async_copy(
                tbl.at[sv.at[pl.ds((g0 * _NBUF + j) * _K, _K)]],
                rows[bank * _NBUF + j], sem_g[bank])

    def fire_s(g0, bank):
        for j in range(_NBUF):
            pltpu.async_copy(
                rows[bank * _NBUF + j],
                agg.at[dv.at[pl.ds((g0 * _NBUF + j) * _K, _K)]],
                sem_s[bank], add=True)

    def drain(sem):
        for j in range(_NBUF):
            pltpu.make_async_copy(zeros_hbm.at[pl.ds(0, _K)], rows[0],
                                  sem).wait()

    for v in range(3):
        tbl = h2_hbm.at[pl.ds((cid * 3 + v) * _N, _N)]
        pltpu.sync_copy(zeros_hbm, agg.at[pl.ds(rbase, _RPT)])
        plsc.subcore_barrier()
        for hh in range(2):
            off = sid * _EPT + hh * (_NBH * _K)
            pltpu.sync_copy(idx_all.at[pl.ds(v * _E + off, _NBH * _K)], sv)
            pltpu.sync_copy(idx_all.at[pl.ds((3 + v) * _E + off,
                                             _NBH * _K)], dv)

            fire_g(tbl, 0, 0)
            fire_g(tbl, 1, 1)

            def pair(t, carry, tbl=tbl):
                g0 = 2 * t
                drain(sem_g[0])
                fire_s(g0, 0)
                drain(sem_g[1])
                drain(sem_s[0])
                fire_g(tbl, g0 + 2, 0)
                fire_s(g0 + 1, 1)
                drain(sem_s[1])

                @pl.when(t < _NPAIR - 1)
                def _():
                    fire_g(tbl, g0 + 3, 1)
                return carry

            lax.fori_loop(0, _NPAIR, pair, 0)
            drain(sem_g[0])
            fire_s(_NG - 1, 0)
            drain(sem_s[0])
        plsc.subcore_barrier()
        pltpu.sync_copy(agg.at[pl.ds(rbase, _RPT)],
                        out_hbm.at[v, pl.ds(rbase, _RPT),
                                   pl.ds(cid * _DH, _DH)])
        plsc.subcore_barrier()


def _finalize(partials, degs, bs):
    blk = 1000

    def body(p_ref, d_ref, b_ref, o_ref):
        acc = jnp.zeros((blk, _D), jnp.float32)
        for v in range(3):
            deg = d_ref[v, :, 0] + d_ref[v, :, 16]
            r = lax.rsqrt(jnp.maximum(deg, 1.0))
            acc += p_ref[v] * r[:, None] + b_ref[v][None, :]
        o_ref[...] = acc * (1.0 / 3.0)

    return pl.pallas_call(
        body,
        grid=(_N // blk,),
        in_specs=[
            pl.BlockSpec((3, blk, _D), lambda i: (0, i, 0)),
            pl.BlockSpec((3, blk, _D), lambda i: (1, i, 0)),
            pl.BlockSpec((3, _D), lambda i: (0, 0)),
        ],
        out_specs=pl.BlockSpec((blk, _D), lambda i: (i, 0)),
        out_shape=jax.ShapeDtypeStruct((_N, _D), jnp.float32),
    )(partials, degs, bs)


def kernel(X, edge_index_view0, edge_index_view1, edge_index_view2,
           W_view0, b_view0, W_view1, b_view1, W_view2, b_view2):
    eis = [edge_index_view0, edge_index_view1, edge_index_view2]
    idx_all = jnp.concatenate([eis[0][0], eis[1][0], eis[2][0],
                               eis[0][1], eis[1][1], eis[2][1]])
    zeros16 = jnp.zeros((_RPT, 16), jnp.float32)
    ones16 = jnp.ones((_K, 16), jnp.float32)
    degs = _deg_kernel(idx_all, zeros16, ones16)

    Ws2 = jnp.stack([jnp.stack([W_view0[:, :_DH], W_view1[:, :_DH],
                                W_view2[:, :_DH]]),
                     jnp.stack([W_view0[:, _DH:], W_view1[:, _DH:],
                                W_view2[:, _DH:]])])
    bs = jnp.stack([b_view0, b_view1, b_view2])
    h2 = _matmul3(X, degs, Ws2)

    zeros64 = jnp.zeros((_RPT, _DH), jnp.float32)
    parts = _scatter_kernel(h2, idx_all, zeros64)

    return _finalize(parts, degs, bs)


# speedup vs baseline: 1.1892x; 1.1892x over previous
import functools

import jax
import jax.numpy as jnp
from jax import lax
from jax.experimental import pallas as pl
from jax.experimental.pallas import tpu as pltpu
from jax.experimental.pallas import tpu_sc as plsc

_N = 10000
_NP = 10240
_E = 320000
_D = 128
_NC = 2
_NS = 16
_NW = _NC * _NS
_EPW = _E // _NW
_EPT = _E // _NS
_K = 80
_NB = _EPW // _K
_NBT = _EPT // _K
_RPT = _NP // _NS
_DH = _D // 2

_mesh = plsc.VectorSubcoreMesh(core_axis_name="c", subcore_axis_name="s")
_sc_params = pltpu.CompilerParams(use_tc_tiling_on_sc=False)


@functools.partial(
    pl.kernel,
    out_type=jax.ShapeDtypeStruct((6, _NP, _D), jnp.float32),
    mesh=_mesh,
    scratch_types=[
        pltpu.VMEM_SHARED((_NP, 16), jnp.float32),
        pltpu.VMEM_SHARED((_NP, 16), jnp.float32),
        pltpu.VMEM((_EPW,), jnp.int32),
        pltpu.VMEM((_K, 16), jnp.float32),
        pltpu.SemaphoreType.DMA,
    ],
    compiler_params=_sc_params,
)
def _deg_kernel(i0, i1, i2, i3, i4, i5, zeros_hbm, ones_hbm, out_hbm,
                sp0, sp1, iv, ones_v, sem):
    cid = lax.axis_index("c")
    sid = lax.axis_index("s")
    ebase = (cid * _NS + sid) * _EPW
    rbase = sid * _RPT
    idxs = [i0, i1, i2, i3, i4, i5]
    sps = [sp0, sp1]
    pltpu.sync_copy(ones_hbm, ones_v)
    for g in range(3):
        for j in range(2):
            pltpu.sync_copy(zeros_hbm, sps[j].at[pl.ds(rbase, _RPT)])
        plsc.subcore_barrier()
        for j in range(2):
            pltpu.sync_copy(idxs[g * 2 + j].at[pl.ds(ebase, _EPW)], iv)

            def fire(b, carry, j=j):
                pltpu.async_copy(ones_v, sps[j].at[iv.at[pl.ds(b * _K, _K)]],
                                 sem, add=True)
                return carry
            lax.fori_loop(0, _NB, fire, 0)

            def drain(b, carry):
                pltpu.make_async_copy(ones_hbm, ones_v, sem).wait()
                return carry
            lax.fori_loop(0, _NB, drain, 0)
        plsc.subcore_barrier()
        for j in range(2):
            pltpu.sync_copy(
                sps[j].at[pl.ds(rbase, _RPT)],
                out_hbm.at[g * 2 + j, pl.ds(rbase, _RPT),
                           pl.ds(cid * 16, 16)])
        plsc.subcore_barrier()


def _rsqrt_table(degs):
    blk = 1000

    def body(d_ref, o_ref):
        cols = []
        for h in range(6):
            deg = d_ref[h, :, 0] + d_ref[h, :, 16]
            cols.append(lax.rsqrt(jnp.maximum(deg, 1.0)))
        o_ref[...] = jnp.stack(cols, axis=1)

    return pl.pallas_call(
        body,
        grid=(_N // blk,),
        in_specs=[pl.BlockSpec((6, blk, _D), lambda i: (0, i, 0))],
        out_specs=pl.BlockSpec((blk, 6), lambda i: (i, 0)),
        out_shape=jax.ShapeDtypeStruct((_N, 6), jnp.float32),
    )(degs)


def _matmul3(X, rs, Ws2):
    blk = 1000

    def body(x_ref, r_ref, w_ref, o_ref):
        x = x_ref[...]
        for v in range(3):
            xs = x * r_ref[:, v][:, None]
            for c in range(2):
                o_ref[c, v] = jnp.dot(xs, w_ref[c, v],
                                      preferred_element_type=jnp.float32)

    return pl.pallas_call(
        body,
        grid=(_N // blk,),
        in_specs=[
            pl.BlockSpec((blk, _D), lambda i: (i, 0)),
            pl.BlockSpec((blk, 6), lambda i: (i, 0)),
            pl.BlockSpec((2, 3, _D, _DH), lambda i: (0, 0, 0, 0)),
        ],
        out_specs=pl.BlockSpec((2, 3, blk, _DH), lambda i: (0, 0, i, 0)),
        out_shape=jax.ShapeDtypeStruct((2, 3, _N, _DH), jnp.float32),
    )(X, rs, Ws2)


_NBUF = 5
_NBH = _NBT // 2
_NG = _NBH // _NBUF
_NPAIR = (_NG - 1) // 2


@functools.partial(
    pl.kernel,
    out_type=jax.ShapeDtypeStruct((3, _NP, _D), jnp.float32),
    mesh=_mesh,
    scratch_types=[
        pltpu.VMEM_SHARED((_NP, _DH), jnp.float32),
        pltpu.VMEM((_NBH * _K,), jnp.int32),
        pltpu.VMEM((_NBH * _K,), jnp.int32),
        [pltpu.VMEM((_K, _DH), jnp.float32) for _ in range(2 * _NBUF)],
        [pltpu.SemaphoreType.DMA for _ in range(4)],
    ],
    compiler_params=_sc_params,
)
def _scatter_kernel(h2_hbm, s0, s1, s2, d0, d1, d2, zeros_hbm, out_hbm,
                    agg, sv, dv, rows, sems):
    cid = lax.axis_index("c")
    sid = lax.axis_index("s")
    rbase = sid * _RPT
    srcs = [s0, s1, s2]
    dsts = [d0, d1, d2]
    sem_g = [sems[0], sems[1]]
    sem_s = [sems[2], sems[3]]

    def fire_g(tbl, g0, bank):
        for j in range(_NBUF):
            pltpu.async_copy(
                tbl.at[sv.at[pl.ds((g0 * _NBUF + j) * _K, _K)]],
                rows[bank * _NBUF + j], sem_g[bank])

    def fire_s(g0, bank):
        for j in range(_NBUF):
            pltpu.async_copy(
                rows[bank * _NBUF + j],
                agg.at[dv.at[pl.ds((g0 * _NBUF + j) * _K, _K)]],
                sem_s[bank], add=True)

    def drain(sem):
        for j in range(_NBUF):
            pltpu.make_async_copy(zeros_hbm.at[pl.ds(0, _K)], rows[0],
                                  sem).wait()

    for v in range(3):
        tbl = h2_hbm.at[pl.ds((cid * 3 + v) * _N, _N)]
        pltpu.sync_copy(zeros_hbm, agg.at[pl.ds(rbase, _RPT)])
        plsc.subcore_barrier()
        for hh in range(2):
            off = sid * _EPT + hh * (_NBH * _K)
            pltpu.sync_copy(srcs[v].at[pl.ds(off, _NBH * _K)], sv)
            pltpu.sync_copy(dsts[v].at[pl.ds(off, _NBH * _K)], dv)

            fire_g(tbl, 0, 0)
            fire_g(tbl, 1, 1)

            def pair(t, carry, tbl=tbl):
                g0 = 2 * t
                drain(sem_g[0])
                fire_s(g0, 0)
                drain(sem_g[1])
                drain(sem_s[0])
                fire_g(tbl, g0 + 2, 0)
                fire_s(g0 + 1, 1)
                drain(sem_s[1])

                @pl.when(t < _NPAIR - 1)
                def _():
                    fire_g(tbl, g0 + 3, 1)
                return carry

            lax.fori_loop(0, _NPAIR, pair, 0)
            drain(sem_g[0])
            fire_s(_NG - 1, 0)
            drain(sem_s[0])
        plsc.subcore_barrier()
        pltpu.sync_copy(agg.at[pl.ds(rbase, _RPT)],
                        out_hbm.at[v, pl.ds(rbase, _RPT),
                                   pl.ds(cid * _DH, _DH)])
        plsc.subcore_barrier()


def _finalize(partials, rs, bs):
    blk = 1000

    def body(p_ref, r_ref, b_ref, o_ref):
        acc = jnp.zeros((blk, _D), jnp.float32)
        for v in range(3):
            r = r_ref[:, 3 + v]
            acc += p_ref[v] * r[:, None] + b_ref[v][None, :]
        o_ref[...] = acc * (1.0 / 3.0)

    return pl.pallas_call(
        body,
        grid=(_N // blk,),
        in_specs=[
            pl.BlockSpec((3, blk, _D), lambda i: (0, i, 0)),
            pl.BlockSpec((blk, 6), lambda i: (i, 0)),
            pl.BlockSpec((3, _D), lambda i: (0, 0)),
        ],
        out_specs=pl.BlockSpec((blk, _D), lambda i: (i, 0)),
        out_shape=jax.ShapeDtypeStruct((_N, _D), jnp.float32),
    )(partials, rs, bs)


def kernel(X, edge_index_view0, edge_index_view1, edge_index_view2,
           W_view0, b_view0, W_view1, b_view1, W_view2, b_view2):
    eis = [edge_index_view0, edge_index_view1, edge_index_view2]
    srcs = [eis[v][0] for v in range(3)]
    dsts = [eis[v][1] for v in range(3)]
    zeros16 = jnp.zeros((_RPT, 16), jnp.float32)
    ones16 = jnp.ones((_K, 16), jnp.float32)
    degs = _deg_kernel(*srcs, *dsts, zeros16, ones16)
    rs = _rsqrt_table(degs)

    Ws2 = jnp.stack([jnp.stack([W_view0[:, :_DH], W_view1[:, :_DH],
                                W_view2[:, :_DH]]),
                     jnp.stack([W_view0[:, _DH:], W_view1[:, _DH:],
                                W_view2[:, _DH:]])])
    bs = jnp.stack([b_view0, b_view1, b_view2])
    h2 = _matmul3(X, rs, Ws2).reshape(2 * 3 * _N, _DH)

    zeros64 = jnp.zeros((_RPT, _DH), jnp.float32)
    parts = _scatter_kernel(h2, *srcs, *dsts, zeros64)

    return _finalize(parts, rs, bs)


# speedup vs baseline: 1.1893x; 1.0001x over previous
import functools

import jax
import jax.numpy as jnp
from jax import lax
from jax.experimental import pallas as pl
from jax.experimental.pallas import tpu as pltpu
from jax.experimental.pallas import tpu_sc as plsc

_N = 10000
_NP = 10240
_E = 320000
_D = 128
_NC = 2
_NS = 16
_NW = _NC * _NS
_EPW = _E // _NW
_EPT = _E // _NS
_K = 80
_NB = _EPW // _K
_NBT = _EPT // _K
_RPT = _NP // _NS
_DH = _D // 2

_mesh = plsc.VectorSubcoreMesh(core_axis_name="c", subcore_axis_name="s")
_sc_params = pltpu.CompilerParams(use_tc_tiling_on_sc=False)


@functools.partial(
    pl.kernel,
    out_type=jax.ShapeDtypeStruct((6, _NP, _D), jnp.float32),
    mesh=_mesh,
    scratch_types=[
        pltpu.VMEM_SHARED((_NP, 16), jnp.float32),
        pltpu.VMEM_SHARED((_NP, 16), jnp.float32),
        pltpu.VMEM((_EPW,), jnp.int32),
        pltpu.VMEM((128, 16), jnp.float32),
        pltpu.SemaphoreType.DMA,
    ],
    compiler_params=_sc_params,
)
def _deg_kernel(i0, i1, i2, i3, i4, i5, zeros_hbm, ones_hbm, out_hbm,
                sp0, sp1, iv, ones_v, sem):
    cid = lax.axis_index("c")
    sid = lax.axis_index("s")
    ebase = (cid * _NS + sid) * _EPW
    rbase = sid * _RPT
    idxs = [i0, i1, i2, i3, i4, i5]
    sps = [sp0, sp1]
    pltpu.sync_copy(ones_hbm, ones_v)
    for g in range(3):
        for j in range(2):
            pltpu.sync_copy(zeros_hbm, sps[j].at[pl.ds(rbase, _RPT)])
        plsc.subcore_barrier()
        for j in range(2):
            pltpu.sync_copy(idxs[g * 2 + j].at[pl.ds(ebase, _EPW)], iv)

            def fire(b, carry, j=j):
                pltpu.async_copy(ones_v,
                                 sps[j].at[iv.at[pl.ds(b * 128, 128)]],
                                 sem, add=True)
                return carry
            lax.fori_loop(0, 78, fire, 0)
            pltpu.async_copy(ones_v.at[pl.ds(0, 16)],
                             sps[j].at[iv.at[pl.ds(78 * 128, 16)]],
                             sem, add=True)

            def drain(b, carry):
                pltpu.make_async_copy(ones_hbm, ones_v, sem).wait()
                return carry
            lax.fori_loop(0, 78, drain, 0)
            pltpu.make_async_copy(ones_hbm.at[pl.ds(0, 16)],
                                  ones_v.at[pl.ds(0, 16)], sem).wait()
        plsc.subcore_barrier()
        for j in range(2):
            pltpu.sync_copy(
                sps[j].at[pl.ds(rbase, _RPT)],
                out_hbm.at[g * 2 + j, pl.ds(rbase, _RPT),
                           pl.ds(cid * 16, 16)])
        plsc.subcore_barrier()


def _rsqrt_table(degs):
    blk = 1000

    def body(d_ref, o_ref):
        cols = []
        for h in range(6):
            deg = d_ref[h, :, 0] + d_ref[h, :, 16]
            cols.append(lax.rsqrt(jnp.maximum(deg, 1.0)))
        o_ref[...] = jnp.stack(cols, axis=1)

    return pl.pallas_call(
        body,
        grid=(_N // blk,),
        in_specs=[pl.BlockSpec((6, blk, _D), lambda i: (0, i, 0))],
        out_specs=pl.BlockSpec((blk, 6), lambda i: (i, 0)),
        out_shape=jax.ShapeDtypeStruct((_N, 6), jnp.float32),
    )(degs)


def _matmul3(X, rs, Ws2):
    blk = 1000

    def body(x_ref, r_ref, w_ref, o_ref):
        x = x_ref[...]
        for v in range(3):
            xs = x * r_ref[:, v][:, None]
            for c in range(2):
                o_ref[c, v] = jnp.dot(xs, w_ref[c, v],
                                      preferred_element_type=jnp.float32)

    return pl.pallas_call(
        body,
        grid=(_N // blk,),
        in_specs=[
            pl.BlockSpec((blk, _D), lambda i: (i, 0)),
            pl.BlockSpec((blk, 6), lambda i: (i, 0)),
            pl.BlockSpec((2, 3, _D, _DH), lambda i: (0, 0, 0, 0)),
        ],
        out_specs=pl.BlockSpec((2, 3, blk, _DH), lambda i: (0, 0, i, 0)),
        out_shape=jax.ShapeDtypeStruct((2, 3, _N, _DH), jnp.float32),
    )(X, rs, Ws2)


_NBUF = 5
_NBH = _NBT // 2
_NG = _NBH // _NBUF
_NPAIR = (_NG - 1) // 2


@functools.partial(
    pl.kernel,
    out_type=jax.ShapeDtypeStruct((3, _NP, _D), jnp.float32),
    mesh=_mesh,
    scratch_types=[
        pltpu.VMEM_SHARED((_NP, _DH), jnp.float32),
        pltpu.VMEM((_NBH * _K,), jnp.int32),
        pltpu.VMEM((_NBH * _K,), jnp.int32),
        [pltpu.VMEM((_K, _DH), jnp.float32) for _ in range(2 * _NBUF)],
        [pltpu.SemaphoreType.DMA for _ in range(4)],
    ],
    compiler_params=_sc_params,
)
def _scatter_kernel(h2_hbm, s0, s1, s2, d0, d1, d2, zeros_hbm, out_hbm,
                    agg, sv, dv, rows, sems):
    cid = lax.axis_index("c")
    sid = lax.axis_index("s")
    rbase = sid * _RPT
    srcs = [s0, s1, s2]
    dsts = [d0, d1, d2]
    sem_g = [sems[0], sems[1]]
    sem_s = [sems[2], sems[3]]

    def fire_g(tbl, g0, bank):
        for j in range(_NBUF):
            pltpu.async_copy(
                tbl.at[sv.at[pl.ds((g0 * _NBUF + j) * _K, _K)]],
                rows[bank * _NBUF + j], sem_g[bank])

    def fire_s(g0, bank):
        for j in range(_NBUF):
            pltpu.async_copy(
                rows[bank * _NBUF + j],
                agg.at[dv.at[pl.ds((g0 * _NBUF + j) * _K, _K)]],
                sem_s[bank], add=True)

    def drain(sem):
        for j in range(_NBUF):
            pltpu.make_async_copy(zeros_hbm.at[pl.ds(0, _K)], rows[0],
                                  sem).wait()

    for v in range(3):
        tbl = h2_hbm.at[pl.ds((cid * 3 + v) * _N, _N)]
        pltpu.sync_copy(zeros_hbm, agg.at[pl.ds(rbase, _RPT)])
        plsc.subcore_barrier()
        for hh in range(2):
            off = sid * _EPT + hh * (_NBH * _K)
            pltpu.sync_copy(srcs[v].at[pl.ds(off, _NBH * _K)], sv)
            pltpu.sync_copy(dsts[v].at[pl.ds(off, _NBH * _K)], dv)

            fire_g(tbl, 0, 0)
            fire_g(tbl, 1, 1)

            def pair(t, carry, tbl=tbl):
                g0 = 2 * t
                drain(sem_g[0])
                fire_s(g0, 0)
                drain(sem_g[1])
                drain(sem_s[0])
                fire_g(tbl, g0 + 2, 0)
                fire_s(g0 + 1, 1)
                drain(sem_s[1])

                @pl.when(t < _NPAIR - 1)
                def _():
                    fire_g(tbl, g0 + 3, 1)
                return carry

            lax.fori_loop(0, _NPAIR, pair, 0)
            drain(sem_g[0])
            fire_s(_NG - 1, 0)
            drain(sem_s[0])
        plsc.subcore_barrier()
        pltpu.sync_copy(agg.at[pl.ds(rbase, _RPT)],
                        out_hbm.at[v, pl.ds(rbase, _RPT),
                                   pl.ds(cid * _DH, _DH)])
        plsc.subcore_barrier()


def _finalize(partials, rs, bs):
    blk = 1000

    def body(p_ref, r_ref, b_ref, o_ref):
        acc = jnp.zeros((blk, _D), jnp.float32)
        for v in range(3):
            r = r_ref[:, 3 + v]
            acc += p_ref[v] * r[:, None] + b_ref[v][None, :]
        o_ref[...] = acc * (1.0 / 3.0)

    return pl.pallas_call(
        body,
        grid=(_N // blk,),
        in_specs=[
            pl.BlockSpec((3, blk, _D), lambda i: (0, i, 0)),
            pl.BlockSpec((blk, 6), lambda i: (i, 0)),
            pl.BlockSpec((3, _D), lambda i: (0, 0)),
        ],
        out_specs=pl.BlockSpec((blk, _D), lambda i: (i, 0)),
        out_shape=jax.ShapeDtypeStruct((_N, _D), jnp.float32),
    )(partials, rs, bs)


def kernel(X, edge_index_view0, edge_index_view1, edge_index_view2,
           W_view0, b_view0, W_view1, b_view1, W_view2, b_view2):
    eis = [edge_index_view0, edge_index_view1, edge_index_view2]
    srcs = [eis[v][0] for v in range(3)]
    dsts = [eis[v][1] for v in range(3)]
    zeros16 = jnp.zeros((_RPT, 16), jnp.float32)
    ones16 = jnp.ones((128, 16), jnp.float32)
    degs = _deg_kernel(*srcs, *dsts, zeros16, ones16)
    rs = _rsqrt_table(degs)

    Ws2 = jnp.stack([jnp.stack([W_view0[:, :_DH], W_view1[:, :_DH],
                                W_view2[:, :_DH]]),
                     jnp.stack([W_view0[:, _DH:], W_view1[:, _DH:],
                                W_view2[:, _DH:]])])
    bs = jnp.stack([b_view0, b_view1, b_view2])
    h2 = _matmul3(X, rs, Ws2).reshape(2 * 3 * _N, _DH)

    zeros64 = jnp.zeros((_RPT, _DH), jnp.float32)
    parts = _scatter_kernel(h2, *srcs, *dsts, zeros64)

    return _finalize(parts, rs, bs)
